# Initial kernel scaffold; baseline (speedup 1.0000x reference)
import jax
import jax.numpy as jnp
from jax.experimental import pallas as pl


def kernel(x, edge_index, W1, b1, g1, be1, W2, b2, g2, be2, fW1, fb1, fW2, fb2):
    raise NotImplementedError("write your pallas kernel here")


# speedup vs baseline: 24.0371x; 24.0371x over previous
import functools

import jax
import jax.numpy as jnp
from jax import lax
from jax.experimental import pallas as pl
from jax.experimental.pallas import tpu as pltpu
from jax.experimental.pallas import tpu_sc as plsc

N = 100000
E = 1600000
F = 32
HH = 16
NC = 2
NS = 16

NPAD = 100352
ROWS_PER_TILE_N = NPAD // NS
DUMMY = NPAD - 1

EB = 128
AGG_BLK = 8
AGG_NBLK = 98
ROWS_PER_TILE_E = AGG_BLK * AGG_NBLK
EROWS = ROWS_PER_TILE_E * NS
EPAD = EROWS * EB
DEG_BLK = 8
DEG_NBLK = 49

_mesh = plsc.VectorSubcoreMesh(
    core_axis_name="c", subcore_axis_name="s", num_cores=NC, num_subcores=NS)


def _zero_spmem(buf_v, sh, s):
  for i in range(EB):
    buf_v[i, :] = jnp.zeros((16,), jnp.float32)

  def zloop(i, carry):
    pltpu.sync_copy(buf_v, sh.at[pl.ds(s * ROWS_PER_TILE_N + i * EB, EB)])
    return carry

  lax.fori_loop(0, ROWS_PER_TILE_N // EB, zloop, 0)


def _deg_body(dst_hbm, deg_hbm, dst_v, ones_v, deg_sh):
  c = lax.axis_index("c")
  s = lax.axis_index("s")
  _zero_spmem(ones_v, deg_sh, s)
  for i in range(EB):
    ones_v[i, :] = jnp.ones((16,), jnp.float32)
  plsc.subcore_barrier()

  def blk(b, carry):
    r0 = c * (EROWS // NC) + s * (DEG_NBLK * DEG_BLK) + b * DEG_BLK
    pltpu.sync_copy(dst_hbm.at[pl.ds(r0, DEG_BLK)], dst_v)
    for j in range(DEG_BLK):
      pltpu.sync_copy(ones_v, deg_sh.at[dst_v.at[j]], add=True)
    return carry

  lax.fori_loop(0, DEG_NBLK, blk, 0)
  plsc.subcore_barrier()
  r = s * ROWS_PER_TILE_N
  pltpu.sync_copy(deg_sh.at[pl.ds(r, ROWS_PER_TILE_N)],
                  deg_hbm.at[pl.ds(c * NPAD + r, ROWS_PER_TILE_N)])


_sc_params = pltpu.CompilerParams(use_tc_tiling_on_sc=False)

_deg_call = functools.partial(
    pl.kernel,
    out_type=jax.ShapeDtypeStruct((NC * NPAD, HH), jnp.float32),
    mesh=_mesh,
    compiler_params=_sc_params,
    scratch_types=[
        pltpu.VMEM((DEG_BLK, EB), jnp.int32),
        pltpu.VMEM((EB, HH), jnp.float32),
        pltpu.VMEM_SHARED((NPAD, HH), jnp.float32),
    ],
)(_deg_body)


def _agg_body(src_hbm, dst_hbm, hs_hbm, acc_hbm,
              src_v, dst_v, rows_v, zero_v, acc_sh, gsem):
  c = lax.axis_index("c")
  s = lax.axis_index("s")
  _zero_spmem(zero_v, acc_sh, s)
  plsc.subcore_barrier()
  coff = c * NPAD

  def blk(b, carry):
    r0 = s * ROWS_PER_TILE_E + b * AGG_BLK
    pltpu.sync_copy(src_hbm.at[pl.ds(r0, AGG_BLK)], src_v)
    pltpu.sync_copy(dst_hbm.at[pl.ds(r0, AGG_BLK)], dst_v)
    for j in range(AGG_BLK):
      for i in range(EB // 16):
        src_v[j, pl.ds(i * 16, 16)] = src_v[j, pl.ds(i * 16, 16)] + coff
    descs = []
    for j in range(AGG_BLK):
      descs.append(
          pltpu.async_copy(hs_hbm.at[src_v.at[j]], rows_v.at[j], gsem))
    for d in descs:
      d.wait()
    for j in range(AGG_BLK):
      pltpu.sync_copy(rows_v.at[j], acc_sh.at[dst_v.at[j]], add=True)
    return carry

  lax.fori_loop(0, AGG_NBLK, blk, 0)
  plsc.subcore_barrier()
  r = s * ROWS_PER_TILE_N
  pltpu.sync_copy(acc_sh.at[pl.ds(r, ROWS_PER_TILE_N)],
                  acc_hbm.at[pl.ds(c * NPAD + r, ROWS_PER_TILE_N)])


_agg_call = functools.partial(
    pl.kernel,
    out_type=jax.ShapeDtypeStruct((NC * NPAD, HH), jnp.float32),
    mesh=_mesh,
    compiler_params=_sc_params,
    scratch_types=[
        pltpu.VMEM((AGG_BLK, EB), jnp.int32),
        pltpu.VMEM((AGG_BLK, EB), jnp.int32),
        pltpu.VMEM((AGG_BLK, EB, HH), jnp.float32),
        pltpu.VMEM((EB, HH), jnp.float32),
        pltpu.VMEM_SHARED((NPAD, HH), jnp.float32),
        pltpu.SemaphoreType.DMA,
    ],
)(_agg_body)


BN_ = 2048
GRID = NPAD // BN_
BN_EPS = 1e-5


def _dinv(degA, degB):
  deg = degA[:, :1] + degB[:, :1] + 1.0
  return lax.rsqrt(deg)


def _tc1_body(x_ref, degA_ref, degB_ref, w1_ref, hs_ref):
  dinv = _dinv(degA_ref[...], degB_ref[...])
  h = jnp.dot(x_ref[...], w1_ref[...], preferred_element_type=jnp.float32)
  hs = h * dinv
  hs_ref[0, :, :] = hs[:, :HH]
  hs_ref[1, :, :] = hs[:, HH:]


def _tc2_body(accA_ref, accB_ref, hsA_ref, hsB_ref, degA_ref, degB_ref,
              w2_ref, b1_ref, g1_ref, be1_ref, hs2_ref):
  dinv = _dinv(degA_ref[...], degB_ref[...])
  gA = dinv * (accA_ref[...] + hsA_ref[...]) + b1_ref[0, :HH]
  gB = dinv * (accB_ref[...] + hsB_ref[...]) + b1_ref[0, HH:]
  inv_std = 1.0 / jnp.sqrt(1.0 + BN_EPS)
  gA = jnp.maximum(gA * inv_std * g1_ref[0, :HH] + be1_ref[0, :HH], 0.0)
  gB = jnp.maximum(gB * inv_std * g1_ref[0, HH:] + be1_ref[0, HH:], 0.0)
  h1 = jnp.concatenate([gA, gB], axis=1)
  h2 = jnp.dot(h1, w2_ref[...], preferred_element_type=jnp.float32)
  hs2 = h2 * dinv
  hs2_ref[0, :, :] = hs2[:, :HH]
  hs2_ref[1, :, :] = hs2[:, HH:]


def _tc3_body(accA_ref, accB_ref, hsA_ref, hsB_ref, degA_ref, degB_ref,
              b2_ref, g2_ref, be2_ref, fw1_ref, fb1_ref, fw2_ref, fb2_ref,
              out_ref):
  dinv = _dinv(degA_ref[...], degB_ref[...])
  gA = dinv * (accA_ref[...] + hsA_ref[...]) + b2_ref[0, :HH]
  gB = dinv * (accB_ref[...] + hsB_ref[...]) + b2_ref[0, HH:]
  inv_std = 1.0 / jnp.sqrt(1.0 + BN_EPS)
  gA = jnp.maximum(gA * inv_std * g2_ref[0, :HH] + be2_ref[0, :HH], 0.0)
  gB = jnp.maximum(gB * inv_std * g2_ref[0, HH:] + be2_ref[0, HH:], 0.0)
  h = jnp.concatenate([gA, gB], axis=1)
  f = jnp.dot(h, fw1_ref[...], preferred_element_type=jnp.float32)
  f = jnp.maximum(f + fb1_ref[0, :], 0.0)
  y = jnp.dot(f, fw2_ref[...], preferred_element_type=jnp.float32)
  y = y + fb2_ref[0, :]
  m = jnp.max(y, axis=1, keepdims=True)
  lse = m + jnp.log(jnp.sum(jnp.exp(y - m), axis=1, keepdims=True))
  out_ref[...] = y - lse


def _nspec(cols, half=0):
  off = half * GRID
  return pl.BlockSpec((BN_, cols), lambda i, o=off: (i + o, 0))


def _fullspec(shape):
  nd = len(shape)
  return pl.BlockSpec(shape, lambda i: (0,) * nd)


def kernel(x, edge_index, W1, b1, g1, be1, W2, b2, g2, be2, fW1, fb1, fW2, fb2):
  f32 = jnp.float32
  src = edge_index[0].astype(jnp.int32)
  dst = edge_index[1].astype(jnp.int32)
  pad = EPAD - E
  srcR = jnp.concatenate([src, jnp.zeros((pad,), jnp.int32)]).reshape(EROWS, EB)
  dstR = jnp.concatenate(
      [dst, jnp.full((pad,), DUMMY, jnp.int32)]).reshape(EROWS, EB)
  x_p = jnp.pad(x, ((0, NPAD - N), (0, 0)))

  b1r = b1.reshape(1, F)
  g1r = g1.reshape(1, F)
  be1r = be1.reshape(1, F)
  b2r = b2.reshape(1, F)
  g2r = g2.reshape(1, F)
  be2r = be2.reshape(1, F)
  fb1r = fb1.reshape(1, 64)
  fb2r = fb2.reshape(1, 2)

  deg2 = _deg_call(dstR)

  hs1 = pl.pallas_call(
      _tc1_body,
      grid=(GRID,),
      in_specs=[
          _nspec(F),
          _nspec(HH, 0), _nspec(HH, 1),
          _fullspec((F, F)),
      ],
      out_specs=pl.BlockSpec((2, BN_, HH), lambda i: (0, i, 0)),
      out_shape=jax.ShapeDtypeStruct((2, NPAD, HH), f32),
  )(x_p, deg2, deg2, W1)
  hs1f = hs1.reshape(2 * NPAD, HH)

  acc1 = _agg_call(srcR, dstR, hs1f)

  hs2 = pl.pallas_call(
      _tc2_body,
      grid=(GRID,),
      in_specs=[
          _nspec(HH, 0), _nspec(HH, 1),
          _nspec(HH, 0), _nspec(HH, 1),
          _nspec(HH, 0), _nspec(HH, 1),
          _fullspec((F, F)),
          _fullspec((1, F)), _fullspec((1, F)), _fullspec((1, F)),
      ],
      out_specs=pl.BlockSpec((2, BN_, HH), lambda i: (0, i, 0)),
      out_shape=jax.ShapeDtypeStruct((2, NPAD, HH), f32),
  )(acc1, acc1, hs1f, hs1f, deg2, deg2, W2, b1r, g1r, be1r)
  hs2f = hs2.reshape(2 * NPAD, HH)

  acc2 = _agg_call(srcR, dstR, hs2f)

  out = pl.pallas_call(
      _tc3_body,
      grid=(GRID,),
      in_specs=[
          _nspec(HH, 0), _nspec(HH, 1),
          _nspec(HH, 0), _nspec(HH, 1),
          _nspec(HH, 0), _nspec(HH, 1),
          _fullspec((1, F)), _fullspec((1, F)), _fullspec((1, F)),
          _fullspec((F, 64)), _fullspec((1, 64)),
          _fullspec((64, 2)), _fullspec((1, 2)),
      ],
      out_specs=pl.BlockSpec((BN_, 2), lambda i: (i, 0)),
      out_shape=jax.ShapeDtypeStruct((NPAD, 2), f32),
  )(acc2, acc2, hs2f, hs2f, deg2, deg2,
    b2r, g2r, be2r, fW1, fb1r, fW2, fb2r)

  return out[:N]
